# baseline (speedup 1.0000x reference)
import functools

import jax
import jax.numpy as jnp
from jax import lax
from jax.experimental import pallas as pl
from jax.experimental.pallas import tpu as pltpu
from jax.experimental.pallas import tpu_sc as plsc

BLK = 256


def _gate_route_body(nexp, nblocks, x_ref, gw_ref, gb_ref,
                     topw_ref, dest_ref, bo_ref,
                     run_ref, eid_s, rank_s):
    pid = pl.program_id(0)

    @pl.when(pid == 0)
    def _():
        run_ref[...] = jnp.zeros_like(run_ref)

    xx = x_ref[...]
    bt = xx.shape[0]
    logits = jnp.dot(xx, gw_ref[...], preferred_element_type=jnp.float32)
    logits = logits + gb_ref[...]
    lanes = lax.broadcasted_iota(jnp.int32, logits.shape, 1)
    logits = jnp.where(lanes < nexp, logits, -jnp.inf)
    m = jnp.max(logits, axis=1, keepdims=True)
    ex = jnp.exp(logits - m)
    s = jnp.sum(ex, axis=1, keepdims=True)
    w = ex / s
    m1 = jnp.max(w, axis=1, keepdims=True)
    i1 = jnp.min(jnp.where(w == m1, lanes, 128), axis=1, keepdims=True)
    w2 = jnp.where(lanes == i1, -1.0, w)
    m2 = jnp.max(w2, axis=1, keepdims=True)
    i2 = jnp.min(jnp.where(w2 == m2, lanes, 128), axis=1, keepdims=True)
    topw_ref[...] = jnp.concatenate([m1, m2], axis=1)
    eid_s[pl.ds(pid * bt, bt), :] = jnp.concatenate([i1, i2], axis=1)

    oh1 = (i1 == lanes).astype(jnp.float32)
    oh2 = (i2 == lanes).astype(jnp.float32)
    O = jnp.concatenate([oh1, oh2], axis=0)
    ba = 2 * bt
    ri = lax.broadcasted_iota(jnp.int32, (ba, ba), 0)
    ci = lax.broadcasted_iota(jnp.int32, (ba, ba), 1)
    tri = (ri > ci).astype(jnp.float32)
    R = jnp.dot(tri, O, preferred_element_type=jnp.float32)
    run = run_ref[...]
    rank_all = jnp.sum(O * (R + run), axis=1, keepdims=True)
    rank_s[pl.ds(pid * bt, bt), :] = jnp.concatenate(
        [rank_all[:bt], rank_all[bt:]], axis=1).astype(jnp.int32)
    csum = jnp.sum(O, axis=0, keepdims=True)
    run_ref[...] = run + csum

    @pl.when(pid == nblocks - 1)
    def _():
        t = eid_s.shape[0]
        c = jnp.where(lanes[:1] < nexp, run + csum, 0.0)
        padded = jnp.ceil(c * (1.0 / BLK)) * BLK
        ri8 = lax.broadcasted_iota(jnp.int32, (128, 128), 0)
        ci8 = lax.broadcasted_iota(jnp.int32, (128, 128), 1)
        triu = (ri8 < ci8).astype(jnp.float32)
        off = jnp.dot(padded, triu, preferred_element_type=jnp.float32)
        lanes_t = lax.broadcasted_iota(jnp.int32, (t, 128), 1)
        cols = []
        for k in range(2):
            ohk = (eid_s[:, k:k + 1] == lanes_t)
            offsel = jnp.sum(jnp.where(ohk, off, 0.0), axis=1, keepdims=True)
            cols.append(rank_s[:, k:k + 1] + offsel.astype(jnp.int32))
        dest_ref[...] = jnp.concatenate(cols, axis=1)
        start = (ri8 * BLK).astype(jnp.float32)
        hit = (start >= off) & (start < off + padded) & (ci8 < nexp)
        bo_ref[...] = jnp.sum(jnp.where(hit, ci8, 0), axis=1, keepdims=True)


def _make_sc_scatter(A, CAP):
    mesh = plsc.VectorSubcoreMesh(core_axis_name="c", subcore_axis_name="s")

    @functools.partial(
        pl.kernel, mesh=mesh,
        out_type=jax.ShapeDtypeStruct((CAP,), jnp.int32),
        scratch_types=[
            pltpu.VMEM((A,), jnp.int32),
            pltpu.VMEM((CAP,), jnp.int32),
        ],
        compiler_params=pltpu.CompilerParams(needs_layout_passes=False),
    )
    def k(dest_hbm, st_hbm, dest_v, st_v):
        cid = lax.axis_index("c")
        sid = lax.axis_index("s")

        @pl.when((cid == 0) & (sid == 0))
        def _():
            pltpu.sync_copy(dest_hbm, dest_v)
            lane = lax.iota(jnp.int32, 16)

            def init_body(i, carry):
                st_v[pl.ds(i * 16, 16)] = (i * 16 + lane) & (A // 2 - 1)
                return carry

            lax.fori_loop(0, CAP // 16, init_body, 0)

            def scat_body(i, carry):
                idx16 = dest_v[pl.ds(i * 16, 16)]
                val16 = lax.shift_right_logical(i * 16 + lane, 1)
                plsc.store_scatter(st_v, [idx16], val16)
                return carry

            lax.fori_loop(0, A // 16, scat_body, 0)
            pltpu.sync_copy(st_v, st_hbm)

    return k


def _make_sc_gather(N, NROWS, nsub, nw, clamp_hi):
    per_w = N // nw
    chunk = 64 if per_w % 64 == 0 else per_w
    nch = per_w // chunk
    mesh = plsc.VectorSubcoreMesh(core_axis_name="c", subcore_axis_name="s")

    @functools.partial(
        pl.kernel, mesh=mesh,
        out_type=jax.ShapeDtypeStruct((N, nsub, 128), jnp.int32),
        scratch_types=[
            pltpu.VMEM((per_w,), jnp.int32),
            pltpu.VMEM((2, chunk, nsub, 128), jnp.int32),
            pltpu.SemaphoreType.DMA,
            pltpu.SemaphoreType.DMA,
            pltpu.SemaphoreType.DMA,
            pltpu.SemaphoreType.DMA,
        ],
    )
    def k(table_hbm, idx_hbm, out_hbm, idx_v, buf_v, g0, g1, w0, w1):
        wid = lax.axis_index("s") * 2 + lax.axis_index("c")
        base = wid * per_w
        pltpu.sync_copy(idx_hbm.at[pl.ds(base, per_w)], idx_v)
        for j in range(per_w // 16):
            v = idx_v[pl.ds(16 * j, 16)]
            idx_v[pl.ds(16 * j, 16)] = jnp.minimum(jnp.maximum(v, 0), clamp_hi)
        gsem = (g0, g1)
        wsem = (w0, w1)

        def start_gather(c):
            return pltpu.async_copy(
                table_hbm.at[idx_v.at[pl.ds(c * chunk, chunk)]],
                buf_v.at[c & 1], gsem[c & 1])

        def start_write(c):
            return pltpu.async_copy(
                buf_v.at[c & 1], out_hbm.at[pl.ds(base + c * chunk, chunk)],
                wsem[c & 1])

        g = {0: start_gather(0)}
        w = {}
        for c in range(nch):
            if c + 1 < nch:
                if c - 1 >= 0:
                    w[c - 1].wait()
                g[c + 1] = start_gather(c + 1)
            g[c].wait()
            w[c] = start_write(c)
        if nch >= 2:
            w[nch - 2].wait()
        w[nch - 1].wait()

    return k


def _gmm_body(owner_ref, xs_ref, ew_ref, eb_ref, out_ref):
    acc = jnp.dot(xs_ref[...], ew_ref[0], preferred_element_type=jnp.float32)
    out_ref[...] = (acc + eb_ref[0]).astype(jnp.bfloat16)


def kernel(x, gate_w, gate_b, expert_w, expert_b):
    B, S, EMB = x.shape
    NE, _, HID = expert_w.shape
    T = B * S
    A = 2 * T
    CAP = A + NE * BLK
    NB = CAP // BLK
    BT1 = min(512, T)

    x2d = x.reshape(T, EMB)
    x16w = lax.bitcast_convert_type(
        x2d.astype(jnp.bfloat16).reshape(T, EMB // 2, 2), jnp.int32
    ).reshape(T, EMB // 256, 128)
    gw = jnp.pad(gate_w, ((0, 0), (0, 128 - NE)))
    gb = jnp.pad(gate_b, (0, 128 - NE)).reshape(1, 128)
    ew16 = expert_w.astype(jnp.bfloat16)

    topw, dest, bo = pl.pallas_call(
        functools.partial(_gate_route_body, NE, T // BT1),
        grid=(T // BT1,),
        in_specs=[
            pl.BlockSpec((BT1, EMB), lambda t: (t, 0)),
            pl.BlockSpec((EMB, 128), lambda t: (0, 0)),
            pl.BlockSpec((1, 128), lambda t: (0, 0)),
        ],
        out_specs=[
            pl.BlockSpec((BT1, 2), lambda t: (t, 0)),
            pl.BlockSpec((T, 2), lambda t: (0, 0)),
            pl.BlockSpec((128, 1), lambda t: (0, 0)),
        ],
        out_shape=[
            jax.ShapeDtypeStruct((T, 2), jnp.float32),
            jax.ShapeDtypeStruct((T, 2), jnp.int32),
            jax.ShapeDtypeStruct((128, 1), jnp.int32),
        ],
        scratch_shapes=[
            pltpu.VMEM((1, 128), jnp.float32),
            pltpu.VMEM((T, 2), jnp.int32),
            pltpu.VMEM((T, 2), jnp.int32),
        ],
    )(x2d, gw, gb)

    dest_flat = dest.reshape(A)
    block_owner = bo.reshape(128)[:NB]

    NW = 32
    st = _make_sc_scatter(A, CAP)(dest_flat)
    xs_w = _make_sc_gather(CAP, T, EMB // 256, NW, T - 1)(x16w, st)
    xs = lax.bitcast_convert_type(
        xs_w.reshape(CAP, EMB // 2), jnp.bfloat16).reshape(CAP, EMB)

    out_sorted = pl.pallas_call(
        _gmm_body,
        grid_spec=pltpu.PrefetchScalarGridSpec(
            num_scalar_prefetch=1,
            grid=(NB,),
            in_specs=[
                pl.BlockSpec((BLK, EMB), lambda g, own: (g, 0)),
                pl.BlockSpec((1, EMB, HID), lambda g, own: (own[g], 0, 0)),
                pl.BlockSpec((1, 1, HID), lambda g, own: (own[g], 0, 0)),
            ],
            out_specs=pl.BlockSpec((BLK, HID), lambda g, own: (g, 0)),
        ),
        out_shape=jax.ShapeDtypeStruct((CAP, HID), jnp.bfloat16),
    )(block_owner, xs, ew16, expert_b.reshape(NE, 1, HID))

    os_w = lax.bitcast_convert_type(
        out_sorted.reshape(CAP, HID // 2, 2), jnp.int32
    ).reshape(CAP, HID // 256, 128)
    out_w = _make_sc_gather(A, CAP, HID // 256, NW, CAP - 1)(os_w, dest_flat)
    out2d = lax.bitcast_convert_type(
        out_w.reshape(A, HID // 2), jnp.bfloat16).reshape(A, HID)

    return (topw.reshape(B, S, 2),
            out2d.astype(jnp.float32).reshape(B, S, 2, HID))


# speedup vs baseline: 21.4039x; 21.4039x over previous
import functools

import jax
import jax.numpy as jnp
from jax import lax
from jax.experimental import pallas as pl


def _fused_body(nexp, sblk, x_ref, gw_ref, gb_ref, ew_ref, eb_ref,
                topw_ref, out_ref):
    xx = x_ref[0]
    bt = xx.shape[0]
    logits = jnp.dot(xx, gw_ref[...], preferred_element_type=jnp.float32)
    logits = logits + gb_ref[...]
    lanes = lax.broadcasted_iota(jnp.int32, logits.shape, 1)
    logits = jnp.where(lanes < nexp, logits, -jnp.inf)
    m = jnp.max(logits, axis=1, keepdims=True)
    ex = jnp.exp(logits - m)
    s = jnp.sum(ex, axis=1, keepdims=True)
    w = ex / s
    m1 = jnp.max(w, axis=1, keepdims=True)
    i1 = jnp.min(jnp.where(w == m1, lanes, 128), axis=1, keepdims=True)
    w2 = jnp.where(lanes == i1, -1.0, w)
    m2 = jnp.max(w2, axis=1, keepdims=True)
    i2 = jnp.min(jnp.where(w2 == m2, lanes, 128), axis=1, keepdims=True)
    topw_ref[0] = jnp.concatenate([m1, m2], axis=1)
    xb = xx.astype(jnp.bfloat16)
    acc1 = jnp.zeros((bt, out_ref.shape[3]), jnp.float32)
    acc2 = jnp.zeros((bt, out_ref.shape[3]), jnp.float32)
    for e in range(nexp):
        oe = jnp.dot(xb, ew_ref[e], preferred_element_type=jnp.float32)
        oe = oe + eb_ref[e][None, :]
        acc1 = jnp.where(i1 == e, oe, acc1)
        acc2 = jnp.where(i2 == e, oe, acc2)
    out_ref[0] = jnp.stack([acc1, acc2], axis=1)


def kernel(x, gate_w, gate_b, expert_w, expert_b):
    B, S, EMB = x.shape
    NE, _, HID = expert_w.shape
    BT = 512
    gw = jnp.pad(gate_w, ((0, 0), (0, 128 - NE)))
    gb = jnp.pad(gate_b, (0, 128 - NE)).reshape(1, 128)
    ew16 = expert_w.astype(jnp.bfloat16)

    topw, out = pl.pallas_call(
        functools.partial(_fused_body, NE, S // BT),
        grid=(B, S // BT),
        in_specs=[
            pl.BlockSpec((1, BT, EMB), lambda b, t: (b, t, 0)),
            pl.BlockSpec((EMB, 128), lambda b, t: (0, 0)),
            pl.BlockSpec((1, 128), lambda b, t: (0, 0)),
            pl.BlockSpec((NE, EMB, HID), lambda b, t: (0, 0, 0)),
            pl.BlockSpec((NE, HID), lambda b, t: (0, 0)),
        ],
        out_specs=[
            pl.BlockSpec((1, BT, 2), lambda b, t: (b, t, 0)),
            pl.BlockSpec((1, BT, 2, HID), lambda b, t: (b, t, 0, 0)),
        ],
        out_shape=[
            jax.ShapeDtypeStruct((B, S, 2), jnp.float32),
            jax.ShapeDtypeStruct((B, S, 2, HID), jnp.float32),
        ],
    )(x, gw, gb, ew16, expert_b)

    return topw, out
